# baseline (speedup 1.0000x reference)
import functools

import jax
import jax.numpy as jnp
from jax import lax
from jax.experimental import pallas as pl
from jax.experimental.pallas import tpu as pltpu
from jax.experimental.pallas import tpu_sc as plsc

BATCH = 4
SEQ = 8192
D = 768

N_ROWS = BATCH * SEQ
NC, NS = 2, 16
NW = NC * NS
K = 4
Q = N_ROWS // K
GW = Q // NW
GC = 64
GN = GW // GC
BLK = 1024


def _gather_body(idx_hbm, table_hbm, out_hbm, idx_v, rows_s, gsems, osems):
    wid = lax.axis_index("s") * NC + lax.axis_index("c")
    base = wid * GW
    pltpu.sync_copy(idx_hbm.at[pl.ds(base, GW)], idx_v)

    def gstart(c, s):
        pltpu.async_copy(
            table_hbm.at[idx_v.at[pl.ds(c * GC, GC)]], rows_s[s], gsems[s])

    def gwait(c, s):
        pltpu.make_async_copy(
            table_hbm.at[idx_v.at[pl.ds(c * GC, GC)]], rows_s[s],
            gsems[s]).wait()

    def ostart(c, s):
        pltpu.async_copy(rows_s[s], out_hbm.at[pl.ds(base + c * GC, GC)],
                         osems[s])

    def owait(c, s):
        pltpu.make_async_copy(
            rows_s[s], out_hbm.at[pl.ds(base + c * GC, GC)], osems[s]).wait()

    gstart(0, 0)
    gstart(1, 1)
    for c in range(GN):
        s = c % 2
        gwait(c, s)
        ostart(c, s)
        nxt = c + 2
        if nxt < GN:
            owait(c, s)
            gstart(nxt, s)
    owait(GN - 2, (GN - 2) % 2)
    owait(GN - 1, (GN - 1) % 2)


@functools.partial(jax.jit, static_argnums=(2,))
def _sc_gather(idx_q, table, tag):
    del tag
    mesh = plsc.VectorSubcoreMesh(core_axis_name="c", subcore_axis_name="s")
    return pl.kernel(
        _gather_body,
        out_type=jax.ShapeDtypeStruct((Q, D), jnp.float32),
        mesh=mesh,
        scratch_types=[
            pltpu.VMEM((GW,), jnp.int32),
            [pltpu.VMEM((GC, D), jnp.float32) for _ in range(2)],
            [pltpu.SemaphoreType.DMA for _ in range(2)],
            [pltpu.SemaphoreType.DMA for _ in range(2)],
        ],
    )(idx_q, table)


def _add_first(x_ref, e_ref, o_ref):
    o_ref[...] = x_ref[...] + e_ref[...]


def _add_upd(prev_ref, x_ref, e_ref, o_ref):
    del prev_ref
    o_ref[...] = x_ref[...] + e_ref[...]


def _tc_add(k, out_prev, x2d, emb):
    nblk = Q // BLK
    off = k * nblk

    xspec = pl.BlockSpec((BLK, D), lambda i: (off + i, 0))
    espec = pl.BlockSpec((BLK, D), lambda i: (i, 0))
    ospec = pl.BlockSpec((BLK, D), lambda i: (off + i, 0))
    oshape = jax.ShapeDtypeStruct((N_ROWS, D), jnp.float32)

    if k == 0:
        return pl.pallas_call(
            _add_first,
            grid=(nblk,),
            in_specs=[xspec, espec],
            out_specs=ospec,
            out_shape=oshape,
        )(x2d, emb)
    return pl.pallas_call(
        _add_upd,
        grid=(nblk,),
        in_specs=[pl.BlockSpec(memory_space=pl.ANY), xspec, espec],
        out_specs=ospec,
        out_shape=oshape,
        input_output_aliases={0: 0},
    )(out_prev, x2d, emb)


@jax.jit
def _run(x2d, idx, table):
    embs = [_sc_gather(lax.dynamic_slice_in_dim(idx, k * Q, Q), table, k)
            for k in range(K)]
    out = None
    for k in range(K):
        out = _tc_add(k, out, x2d, embs[k])
    return out


def kernel(x, position_ids, embeddings):
    x2d = x.reshape(N_ROWS, D)
    idx = position_ids.astype(jnp.int32).reshape(N_ROWS)
    out = _run(x2d, idx, embeddings)
    return out.reshape(BATCH, SEQ, D)


# speedup vs baseline: 1.0067x; 1.0067x over previous
import functools

import jax
import jax.numpy as jnp
from jax import lax
from jax.experimental import pallas as pl
from jax.experimental.pallas import tpu as pltpu
from jax.experimental.pallas import tpu_sc as plsc

BATCH = 4
SEQ = 8192
D = 768

N_ROWS = BATCH * SEQ
NC, NS = 2, 16
NW = NC * NS
K = 2
Q = N_ROWS // K
GW = Q // NW
GC = 64
GN = GW // GC
BLK = 1024


def _gather_body(idx_hbm, table_hbm, out_hbm, idx_v, rows_s, gsems, osems):
    wid = lax.axis_index("s") * NC + lax.axis_index("c")
    base = wid * GW
    pltpu.sync_copy(idx_hbm.at[pl.ds(base, GW)], idx_v)

    def gstart(c, s):
        pltpu.async_copy(
            table_hbm.at[idx_v.at[pl.ds(c * GC, GC)]], rows_s[s], gsems[s])

    def gwait(c, s):
        pltpu.make_async_copy(
            table_hbm.at[idx_v.at[pl.ds(c * GC, GC)]], rows_s[s],
            gsems[s]).wait()

    def ostart(c, s):
        pltpu.async_copy(rows_s[s], out_hbm.at[pl.ds(base + c * GC, GC)],
                         osems[s])

    def owait(c, s):
        pltpu.make_async_copy(
            rows_s[s], out_hbm.at[pl.ds(base + c * GC, GC)], osems[s]).wait()

    gstart(0, 0)
    gstart(1, 1)
    for c in range(GN):
        s = c % 2
        gwait(c, s)
        ostart(c, s)
        nxt = c + 2
        if nxt < GN:
            owait(c, s)
            gstart(nxt, s)
    owait(GN - 2, (GN - 2) % 2)
    owait(GN - 1, (GN - 1) % 2)


@functools.partial(jax.jit, static_argnums=(2,))
def _sc_gather(idx_q, table, tag):
    del tag
    mesh = plsc.VectorSubcoreMesh(core_axis_name="c", subcore_axis_name="s")
    return pl.kernel(
        _gather_body,
        out_type=jax.ShapeDtypeStruct((Q, D), jnp.float32),
        mesh=mesh,
        scratch_types=[
            pltpu.VMEM((GW,), jnp.int32),
            [pltpu.VMEM((GC, D), jnp.float32) for _ in range(2)],
            [pltpu.SemaphoreType.DMA for _ in range(2)],
            [pltpu.SemaphoreType.DMA for _ in range(2)],
        ],
    )(idx_q, table)


def _add_first(x_ref, e_ref, o_ref):
    o_ref[...] = x_ref[...] + e_ref[...]


def _add_upd(prev_ref, x_ref, e_ref, o_ref):
    del prev_ref
    o_ref[...] = x_ref[...] + e_ref[...]


def _tc_add(k, out_prev, x2d, emb):
    nblk = Q // BLK
    off = k * nblk

    xspec = pl.BlockSpec((BLK, D), lambda i: (off + i, 0))
    espec = pl.BlockSpec((BLK, D), lambda i: (i, 0))
    ospec = pl.BlockSpec((BLK, D), lambda i: (off + i, 0))
    oshape = jax.ShapeDtypeStruct((N_ROWS, D), jnp.float32)

    if k == 0:
        return pl.pallas_call(
            _add_first,
            grid=(nblk,),
            in_specs=[xspec, espec],
            out_specs=ospec,
            out_shape=oshape,
        )(x2d, emb)
    return pl.pallas_call(
        _add_upd,
        grid=(nblk,),
        in_specs=[pl.BlockSpec(memory_space=pl.ANY), xspec, espec],
        out_specs=ospec,
        out_shape=oshape,
        input_output_aliases={0: 0},
    )(out_prev, x2d, emb)


@jax.jit
def _run(x2d, idx, table):
    embs = [_sc_gather(lax.dynamic_slice_in_dim(idx, k * Q, Q), table, k)
            for k in range(K)]
    out = None
    for k in range(K):
        out = _tc_add(k, out, x2d, embs[k])
    return out


def kernel(x, position_ids, embeddings):
    x2d = x.reshape(N_ROWS, D)
    idx = position_ids.astype(jnp.int32).reshape(N_ROWS)
    out = _run(x2d, idx, embeddings)
    return out.reshape(BATCH, SEQ, D)
